# Initial kernel scaffold; baseline (speedup 1.0000x reference)
import jax
import jax.numpy as jnp
from jax.experimental import pallas as pl


def kernel(img_left, img_right, dsp_left, dsp_right, mask_left, mask_right):
    raise NotImplementedError("write your pallas kernel here")


# speedup vs baseline: 235.2136x; 235.2136x over previous
import numpy as np
import jax
import jax.numpy as jnp
from jax.experimental import pallas as pl

_H = 512
_OUT = 384
_LO = 128
_SCALE = 1.25


def _bilinear_mat():
    inv = _H / (_H * _SCALE)
    o = np.arange(_LO, _LO + _OUT, dtype=np.float64)
    s = (o + 0.5) * inv - 0.5
    k = np.arange(_H, dtype=np.float64)
    w = np.maximum(0.0, 1.0 - np.abs(s[:, None] - k[None, :]))
    w = w / w.sum(1, keepdims=True)
    return w.astype(np.float32)


def _select_mat():
    src = np.arange(_H)
    tgt = np.round(src.astype(np.float32) * np.float32(_SCALE)).astype(np.int64)
    sel = np.zeros((_OUT, _H), np.float32)
    r = tgt - _LO
    ok = (r >= 0) & (r < _OUT)
    sel[r[ok], src[ok]] = 1.0
    return sel


_R = _bilinear_mat()
_S = _select_mat()


def _body(ximg_ref, xdsp_ref, xm_ref, r_ref, s_ref, oimg_ref, odsp_ref, om_ref):
    r = r_ref[...]
    rt = r_ref[...].T
    for p in range(6):
        t = jnp.dot(r, ximg_ref[p], preferred_element_type=jnp.float32)
        oimg_ref[p] = jnp.dot(t, rt, preferred_element_type=jnp.float32)
    s = s_ref[...]
    st = s_ref[...].T
    for side in range(2):
        m = xm_ref[side]
        for c in range(2):
            fm = xdsp_ref[side * 2 + c] * m * _SCALE
            t = jnp.dot(s, fm, preferred_element_type=jnp.float32)
            odsp_ref[side * 2 + c] = jnp.dot(t, st, preferred_element_type=jnp.float32)
        tm = jnp.dot(s, m, preferred_element_type=jnp.float32)
        om_ref[side] = jnp.dot(tm, st, preferred_element_type=jnp.float32)


def kernel(img_left, img_right, dsp_left, dsp_right, mask_left, mask_right):
    ximg = jnp.concatenate([img_left, img_right], axis=0)
    xdsp = jnp.concatenate([dsp_left, dsp_right], axis=0)
    xm = jnp.stack([mask_left, mask_right]).astype(jnp.float32)

    oimg, odsp, om = pl.pallas_call(
        _body,
        out_shape=(
            jax.ShapeDtypeStruct((6, _OUT, _OUT), jnp.float32),
            jax.ShapeDtypeStruct((4, _OUT, _OUT), jnp.float32),
            jax.ShapeDtypeStruct((2, _OUT, _OUT), jnp.float32),
        ),
    )(ximg, xdsp, xm, jnp.asarray(_R), jnp.asarray(_S))

    return (oimg[:3], oimg[3:], odsp[:2], odsp[2:], om[0], om[1])


# speedup vs baseline: 440.3478x; 1.8721x over previous
import numpy as np
import jax
import jax.numpy as jnp
from jax.experimental import pallas as pl

_H = 512
_OUT = 384
_LO = 128
_SCALE = 1.25


def _bilinear_mat():
    inv = _H / (_H * _SCALE)
    o = np.arange(_LO, _LO + _OUT, dtype=np.float64)
    s = (o + 0.5) * inv - 0.5
    k = np.arange(_H, dtype=np.float64)
    w = np.maximum(0.0, 1.0 - np.abs(s[:, None] - k[None, :]))
    w = w / w.sum(1, keepdims=True)
    return w.astype(np.float32)


def _select_mat():
    src = np.arange(_H)
    tgt = np.round(src.astype(np.float32) * np.float32(_SCALE)).astype(np.int64)
    sel = np.zeros((_OUT, _H), np.float32)
    r = tgt - _LO
    ok = (r >= 0) & (r < _OUT)
    sel[r[ok], src[ok]] = 1.0
    return sel


_R = _bilinear_mat()
_S = _select_mat()


def _body(il_ref, ir_ref, dl_ref, dr_ref, ml_ref, mr_ref, r_ref, s_ref,
          oil_ref, oir_ref, odl_ref, odr_ref, oml_ref, omr_ref):
    r = r_ref[...]
    rt = r_ref[...].T
    for x_ref, o_ref in ((il_ref, oil_ref), (ir_ref, oir_ref)):
        for p in range(3):
            t = jnp.dot(r, x_ref[p], preferred_element_type=jnp.float32)
            o_ref[p] = jnp.dot(t, rt, preferred_element_type=jnp.float32)
    s = s_ref[...]
    st = s_ref[...].T
    for m_ref, d_ref, od_ref, om_ref in (
            (ml_ref, dl_ref, odl_ref, oml_ref),
            (mr_ref, dr_ref, odr_ref, omr_ref)):
        m = m_ref[...].astype(jnp.float32)
        for c in range(2):
            fm = d_ref[c] * m * _SCALE
            t = jnp.dot(s, fm, preferred_element_type=jnp.float32)
            od_ref[c] = jnp.dot(t, st, preferred_element_type=jnp.float32)
        tm = jnp.dot(s, m, preferred_element_type=jnp.float32)
        om_ref[...] = jnp.dot(tm, st, preferred_element_type=jnp.float32)


def kernel(img_left, img_right, dsp_left, dsp_right, mask_left, mask_right):
    return pl.pallas_call(
        _body,
        out_shape=(
            jax.ShapeDtypeStruct((3, _OUT, _OUT), jnp.float32),
            jax.ShapeDtypeStruct((3, _OUT, _OUT), jnp.float32),
            jax.ShapeDtypeStruct((2, _OUT, _OUT), jnp.float32),
            jax.ShapeDtypeStruct((2, _OUT, _OUT), jnp.float32),
            jax.ShapeDtypeStruct((_OUT, _OUT), jnp.float32),
            jax.ShapeDtypeStruct((_OUT, _OUT), jnp.float32),
        ),
    )(img_left, img_right, dsp_left, dsp_right, mask_left, mask_right,
      jnp.asarray(_R), jnp.asarray(_S))
